# Initial kernel scaffold; baseline (speedup 1.0000x reference)
import jax
import jax.numpy as jnp
from jax.experimental import pallas as pl


def kernel(node_features, edge_features, senders, receivers, We1, be1, We2, be2, ln_e_scale, ln_e_bias, Wn1, bn1, Wn2, bn2, ln_n_scale, ln_n_bias):
    raise NotImplementedError("write your pallas kernel here")


# speedup vs baseline: 3.4984x; 3.4984x over previous
import functools

import jax
import jax.numpy as jnp
from jax import lax
from jax.experimental import pallas as pl
from jax.experimental.pallas import tpu as pltpu
from jax.experimental.pallas import tpu_sc as plsc

F32 = jnp.float32
CHUNK = 128
NW = 32


def _proj_body(nf_ref, ws_ref, wr_ref, ps_ref, pr_ref):
    nf = nf_ref[...]
    ps_ref[...] = jnp.dot(nf, ws_ref[...], preferred_element_type=F32)
    pr_ref[...] = jnp.dot(nf, wr_ref[...], preferred_element_type=F32)


def _edge_body(ef_ref, gs_ref, gr_ref, we_ref, be1_ref, we2_ref, be2_ref,
               sc_ref, bi_ref, ue_ref, ne_ref):
    ef = ef_ref[...]
    x = (gs_ref[...] + gr_ref[...] + be1_ref[...]
         + jnp.dot(ef, we_ref[...], preferred_element_type=F32))
    h = jnp.maximum(x, 0.0)
    y = jnp.dot(h, we2_ref[...], preferred_element_type=F32) + be2_ref[...]
    mean = jnp.mean(y, axis=-1, keepdims=True)
    var = jnp.mean((y - mean) ** 2, axis=-1, keepdims=True)
    ue = (y - mean) / jnp.sqrt(var + 1e-5) * sc_ref[...] + bi_ref[...]
    ue_ref[...] = ue
    ne_ref[...] = ue + ef


def _node_body(nf_ref, p0_ref, p1_ref, wa_ref, wb_ref, bn1_ref, wn2_ref,
               bn2_ref, sc_ref, bi_ref, out_ref):
    nf = nf_ref[...]
    agg = p0_ref[...] + p1_ref[...]
    x = (jnp.dot(nf, wa_ref[...], preferred_element_type=F32)
         + jnp.dot(agg, wb_ref[...], preferred_element_type=F32)
         + bn1_ref[...])
    h = jnp.maximum(x, 0.0)
    y = jnp.dot(h, wn2_ref[...], preferred_element_type=F32) + bn2_ref[...]
    mean = jnp.mean(y, axis=-1, keepdims=True)
    var = jnp.mean((y - mean) ** 2, axis=-1, keepdims=True)
    out_ref[...] = ((y - mean) / jnp.sqrt(var + 1e-5) * sc_ref[...]
                    + bi_ref[...] + nf)


def _tc_proj(nf, ws, wr):
    n, d = nf.shape
    bn = 2000
    return pl.pallas_call(
        _proj_body,
        grid=(n // bn,),
        in_specs=[
            pl.BlockSpec((bn, d), lambda i: (i, 0)),
            pl.BlockSpec((d, d), lambda i: (0, 0)),
            pl.BlockSpec((d, d), lambda i: (0, 0)),
        ],
        out_specs=(pl.BlockSpec((bn, d), lambda i: (i, 0)),
                   pl.BlockSpec((bn, d), lambda i: (i, 0))),
        out_shape=(jax.ShapeDtypeStruct((n, d), F32),
                   jax.ShapeDtypeStruct((n, d), F32)),
    )(nf, ws, wr)


def _tc_edge(ef, gs, gr, we, be1, we2, be2, sc, bi):
    e, d = ef.shape
    be = 2000
    row = lambda i: (i, 0)
    cst = lambda i: (0, 0)
    return pl.pallas_call(
        _edge_body,
        grid=(e // be,),
        in_specs=[
            pl.BlockSpec((be, d), row),
            pl.BlockSpec((be, d), row),
            pl.BlockSpec((be, d), row),
            pl.BlockSpec((d, d), cst),
            pl.BlockSpec((1, d), cst),
            pl.BlockSpec((d, d), cst),
            pl.BlockSpec((1, d), cst),
            pl.BlockSpec((1, d), cst),
            pl.BlockSpec((1, d), cst),
        ],
        out_specs=(pl.BlockSpec((be, d), row), pl.BlockSpec((be, d), row)),
        out_shape=(jax.ShapeDtypeStruct((e, d), F32),
                   jax.ShapeDtypeStruct((e, d), F32)),
    )(ef, gs, gr, we, be1, we2, be2, sc, bi)


def _tc_node(nf, p0, p1, wa, wb, bn1, wn2, bn2, sc, bi):
    n, d = nf.shape
    bn = 2000
    row = lambda i: (i, 0)
    cst = lambda i: (0, 0)
    return pl.pallas_call(
        _node_body,
        grid=(n // bn,),
        in_specs=[
            pl.BlockSpec((bn, d), row),
            pl.BlockSpec((bn, d), row),
            pl.BlockSpec((bn, d), row),
            pl.BlockSpec((d, d), cst),
            pl.BlockSpec((d, d), cst),
            pl.BlockSpec((1, d), cst),
            pl.BlockSpec((d, d), cst),
            pl.BlockSpec((1, d), cst),
            pl.BlockSpec((1, d), cst),
            pl.BlockSpec((1, d), cst),
        ],
        out_specs=pl.BlockSpec((bn, d), row),
        out_shape=jax.ShapeDtypeStruct((n, d), F32),
    )(nf, p0, p1, wa, wb, bn1, wn2, bn2, sc, bi)


@functools.lru_cache(maxsize=None)
def _make_gather(n_chunks, d):
    mesh = plsc.VectorSubcoreMesh(core_axis_name="c", subcore_axis_name="s")
    e = n_chunks * CHUNK
    n_iter = (n_chunks + NW - 1) // NW

    @functools.partial(
        pl.kernel,
        out_type=(jax.ShapeDtypeStruct((e, d), F32),
                  jax.ShapeDtypeStruct((e, d), F32)),
        mesh=mesh,
        scratch_types=[
            pltpu.VMEM((CHUNK,), jnp.int32),
            pltpu.VMEM((CHUNK,), jnp.int32),
            pltpu.VMEM((CHUNK, d), F32),
            pltpu.VMEM((CHUNK, d), F32),
            pltpu.SemaphoreType.DMA,
            pltpu.SemaphoreType.DMA,
        ],
    )
    def gather_k(ps_hbm, pr_hbm, sidx_hbm, ridx_hbm, gs_hbm, gr_hbm,
                 sidx_v, ridx_v, rows_a, rows_b, sem_a, sem_b):
        wid = lax.axis_index("s") * 2 + lax.axis_index("c")

        def body(j, carry):
            c = j * NW + wid

            @pl.when(c < n_chunks)
            def _():
                pltpu.sync_copy(sidx_hbm.at[pl.ds(c * CHUNK, CHUNK)], sidx_v)
                pltpu.sync_copy(ridx_hbm.at[pl.ds(c * CHUNK, CHUNK)], ridx_v)
                cp_a = pltpu.async_copy(ps_hbm.at[sidx_v], rows_a, sem_a)
                cp_b = pltpu.async_copy(pr_hbm.at[ridx_v], rows_b, sem_b)
                cp_a.wait()
                cp_b.wait()
                pltpu.sync_copy(rows_a, gs_hbm.at[pl.ds(c * CHUNK, CHUNK)])
                pltpu.sync_copy(rows_b, gr_hbm.at[pl.ds(c * CHUNK, CHUNK)])

            return carry

        lax.fori_loop(0, n_iter, body, 0)

    return gather_k


@functools.lru_cache(maxsize=None)
def _make_scatter(n_nodes, n_chunks, d):
    mesh = plsc.VectorSubcoreMesh(core_axis_name="c", subcore_axis_name="s")
    n_iter = (n_chunks + NW - 1) // NW
    n_pad = ((n_nodes + 16 * 128 - 1) // (16 * 128)) * 16 * 128
    rows_per_tile = n_pad // 16
    zr = 128
    n_zcopy = rows_per_tile // zr

    @functools.partial(
        pl.kernel,
        out_type=jax.ShapeDtypeStruct((2, n_pad, d), F32),
        mesh=mesh,
        scratch_types=[
            pltpu.VMEM((CHUNK,), jnp.int32),
            pltpu.VMEM((CHUNK, d), F32),
            pltpu.VMEM((zr, d), F32),
            pltpu.VMEM_SHARED((n_pad, d), F32),
        ],
    )
    def scatter_k(ue_hbm, ridx_hbm, out_hbm, idx_v, rows_v, zbuf, acc_sh):
        cid = lax.axis_index("c")
        sid = lax.axis_index("s")
        wid = sid * 2 + cid

        def zb(i, carry):
            r = i // (d // 16)
            q = (i % (d // 16)) * 16
            zbuf[r, pl.ds(q, 16)] = jnp.zeros((16,), F32)
            return carry

        lax.fori_loop(0, zr * (d // 16), zb, 0)
        base = sid * rows_per_tile
        for t in range(n_zcopy):
            pltpu.sync_copy(zbuf, acc_sh.at[pl.ds(base + t * zr, zr)])
        plsc.subcore_barrier()

        def body(j, carry):
            c = j * NW + wid

            @pl.when(c < n_chunks)
            def _():
                pltpu.sync_copy(ridx_hbm.at[pl.ds(c * CHUNK, CHUNK)], idx_v)
                pltpu.sync_copy(ue_hbm.at[pl.ds(c * CHUNK, CHUNK)], rows_v)
                pltpu.sync_copy(rows_v, acc_sh.at[idx_v], add=True)

            return carry

        lax.fori_loop(0, n_iter, body, 0)
        plsc.subcore_barrier()
        for t in range(n_zcopy):
            sl = pl.ds(base + t * zr, zr)
            pltpu.sync_copy(acc_sh.at[sl], out_hbm.at[cid, sl])

    return scatter_k


def kernel(node_features, edge_features, senders, receivers,
           We1, be1, We2, be2, ln_e_scale, ln_e_bias,
           Wn1, bn1, Wn2, bn2, ln_n_scale, ln_n_bias):
    n, d = node_features.shape
    e = edge_features.shape[0]
    n_chunks = e // CHUNK

    sidx = senders.astype(jnp.int32)
    ridx = receivers.astype(jnp.int32)

    ws, wr, we = We1[:d], We1[d:2 * d], We1[2 * d:]
    ps, pr = _tc_proj(node_features, ws, wr)
    gs, gr = _make_gather(n_chunks, d)(ps, pr, sidx, ridx)
    ue, ne = _tc_edge(edge_features, gs, gr, we,
                      be1.reshape(1, d), We2, be2.reshape(1, d),
                      ln_e_scale.reshape(1, d), ln_e_bias.reshape(1, d))
    agg2 = _make_scatter(n, n_chunks, d)(ue, ridx)
    new_nodes = _tc_node(node_features, agg2[0, :n], agg2[1, :n],
                         Wn1[:d], Wn1[d:], bn1.reshape(1, d),
                         Wn2, bn2.reshape(1, d),
                         ln_n_scale.reshape(1, d), ln_n_bias.reshape(1, d))
    return (new_nodes, ne)
